# baseline (speedup 1.0000x reference)
import jax
import jax.numpy as jnp
from jax.experimental import pallas as pl


def _dpmean_kernel(f_ref, mu_ref, dist_ref, idx_ref, maxd_ref):
    f = f_ref[...]
    m = mu_ref[...]
    f2 = jnp.sum(f * f, axis=1, keepdims=True)
    mu2 = jnp.sum(m * m, axis=1)
    dot = jax.lax.dot_general(
        f, m, (((1,), (1,)), ((), ())),
        preferred_element_type=jnp.float32)
    d2 = f2 + mu2[None, :] - 2.0 * dot
    d2 = jnp.maximum(d2, 0.0)
    mind2 = jnp.min(d2, axis=1, keepdims=True)
    k = d2.shape[1]
    iota = jax.lax.broadcasted_iota(jnp.int32, d2.shape, 1)
    idx = jnp.min(jnp.where(d2 == mind2, iota, k), axis=1, keepdims=True)
    dist = jnp.sqrt(mind2)
    dist_ref[...] = dist
    idx_ref[...] = idx
    maxd_ref[...] = jnp.max(dist, axis=0, keepdims=True)


def kernel(features, mu):
    f = features[:, 0, :]
    b = f.shape[0]
    dist, idx, maxd = pl.pallas_call(
        _dpmean_kernel,
        out_shape=[
            jax.ShapeDtypeStruct((b, 1), jnp.float32),
            jax.ShapeDtypeStruct((b, 1), jnp.int32),
            jax.ShapeDtypeStruct((1, 1), jnp.float32),
        ],
    )(f, mu)
    return dist[:, 0], idx[:, 0], maxd[0]
